# baseline (speedup 1.0000x reference)
import functools

import jax
import jax.numpy as jnp
from jax import lax
from jax.experimental import pallas as pl
from jax.experimental.pallas import tpu as pltpu

HIDDEN = 1024
FFN = 1024
E = 64
TOPK = 2
T = 4096

S = T * TOPK
BS = 128
S_MAX = S + E * BS
NB = S_MAX // BS
BT = 512

_INTERPRET = False


def _router_body(x_ref, gw_ref, logits_ref, ids_ref, wts_ref):
    x = x_ref[...]
    logits = lax.dot_general(x, gw_ref[...], (((1,), (1,)), ((), ())),
                             preferred_element_type=jnp.float32)
    logits_ref[...] = logits
    m = jnp.max(logits, axis=-1, keepdims=True)
    p = jnp.exp(logits - m)
    p = p / jnp.sum(p, axis=-1, keepdims=True)
    iot = lax.broadcasted_iota(jnp.int32, p.shape, 1)
    m0 = jnp.max(p, axis=-1, keepdims=True)
    i0 = jnp.min(jnp.where(p == m0, iot, E), axis=-1, keepdims=True)
    p2 = jnp.where(iot == i0, -jnp.inf, p)
    m1 = jnp.max(p2, axis=-1, keepdims=True)
    i1 = jnp.min(jnp.where(p2 == m1, iot, E), axis=-1, keepdims=True)
    s = m0 + m1
    ids_ref[...] = jnp.concatenate([i0, i1], axis=-1)
    wts_ref[...] = jnp.concatenate([m0 / s, m1 / s], axis=-1)


def _router(hidden_states, gate_w):
    return pl.pallas_call(
        _router_body,
        grid=(T // BT,),
        in_specs=[
            pl.BlockSpec((BT, HIDDEN), lambda i: (i, 0)),
            pl.BlockSpec((E, HIDDEN), lambda i: (0, 0)),
        ],
        out_specs=[
            pl.BlockSpec((BT, E), lambda i: (i, 0)),
            pl.BlockSpec((BT, TOPK), lambda i: (i, 0)),
            pl.BlockSpec((BT, TOPK), lambda i: (i, 0)),
        ],
        out_shape=[
            jax.ShapeDtypeStruct((T, E), jnp.float32),
            jax.ShapeDtypeStruct((T, TOPK), jnp.int32),
            jax.ShapeDtypeStruct((T, TOPK), jnp.float32),
        ],
        interpret=_INTERPRET,
    )(hidden_states, gate_w)


def _dispatch_indices(ids, wts):
    i32 = jnp.int32
    e_flat = ids.reshape(-1).astype(i32)
    w_flat = wts.reshape(-1)
    order = jnp.argsort(e_flat, stable=True)
    counts = jnp.zeros((E,), i32).at[e_flat].add(1)
    start = jnp.cumsum(counts) - counts
    padded = ((counts + BS - 1) // BS) * BS
    pend = jnp.cumsum(padded)
    poff = pend - padded
    nr = (pend[-1] // BS).astype(i32)

    bstarts = jnp.arange(NB, dtype=i32) * BS
    be_raw = jnp.minimum(jnp.searchsorted(pend, bstarts, side='right'),
                         E - 1).astype(i32)
    be = jnp.where(jnp.arange(NB, dtype=i32) < nr, be_raw,
                   be_raw[jnp.maximum(nr - 1, 0)])

    p = jnp.arange(S_MAX, dtype=i32)
    ep = jnp.minimum(jnp.searchsorted(pend, p, side='right'), E - 1)
    r = p - poff[ep]
    valid = r < counts[ep]
    srck = jnp.clip(start[ep] + r, 0, S - 1)
    src = order[srck]
    tok_p = jnp.where(valid, src // TOPK, 0).astype(i32)
    wt_p = jnp.where(valid, w_flat[src], 0.0)

    ranks = jnp.zeros((S,), i32).at[order].set(jnp.arange(S, dtype=i32))
    dest = poff[e_flat] + (ranks - start[e_flat])
    pos = dest.reshape(T, TOPK)
    return tok_p, wt_p, be, nr, pos[:, 0], pos[:, 1]


def _ffn_body(be_ref, nr_ref, xs_ref, w1_ref, w2_ref, wc_ref, ys_ref):
    i = pl.program_id(0)

    @pl.when(i < nr_ref[0])
    def _():
        x = xs_ref[...].astype(jnp.bfloat16)
        gu = lax.dot_general(x, w1_ref[0].astype(jnp.bfloat16),
                             (((1,), (1,)), ((), ())),
                             preferred_element_type=jnp.float32)
        g = gu[:, :FFN]
        u = gu[:, FFN:]
        h = g * jax.nn.sigmoid(g) * u
        hw = (h * wc_ref[:, 0:1]).astype(jnp.bfloat16)
        ys_ref[...] = lax.dot_general(hw, w2_ref[0].astype(jnp.bfloat16),
                                      (((1,), (1,)), ((), ())),
                                      preferred_element_type=jnp.float32)


def _ffn(xs, w1, w2, wcol, be, nr):
    grid_spec = pltpu.PrefetchScalarGridSpec(
        num_scalar_prefetch=2,
        grid=(NB,),
        in_specs=[
            pl.BlockSpec((BS, HIDDEN), lambda i, be, nr: (i, 0)),
            pl.BlockSpec((1, 2 * FFN, HIDDEN), lambda i, be, nr: (be[i], 0, 0)),
            pl.BlockSpec((1, HIDDEN, FFN), lambda i, be, nr: (be[i], 0, 0)),
            pl.BlockSpec((BS, 128), lambda i, be, nr: (i, 0)),
        ],
        out_specs=pl.BlockSpec((BS, HIDDEN), lambda i, be, nr: (i, 0)),
    )
    return pl.pallas_call(
        _ffn_body,
        grid_spec=grid_spec,
        out_shape=jax.ShapeDtypeStruct((S_MAX, HIDDEN), jnp.float32),
        interpret=_INTERPRET,
    )(be, nr, xs, w1, w2, wcol)


def kernel(hidden_states, gate_w, w1, w2):
    router_logits, ids, wts = _router(hidden_states, gate_w)
    tok_p, wt_p, be, nr, pos0, pos1 = _dispatch_indices(ids, wts)

    xs = hidden_states[tok_p]
    wcol = jnp.broadcast_to(wt_p[:, None], (S_MAX, 128))

    ysw = _ffn(xs, w1, w2, wcol, be, nr[None])

    final = ysw[pos0] + ysw[pos1]
    return final, router_logits


# speedup vs baseline: 2.4663x; 2.4663x over previous
import functools

import jax
import jax.numpy as jnp
from jax import lax
from jax.experimental import pallas as pl
from jax.experimental.pallas import tpu as pltpu

HIDDEN = 1024
FFN = 1024
E = 64
TOPK = 2
T = 4096

S = T * TOPK
BS = 128
S_MAX = S + E * BS
NB = S_MAX // BS
BT = 512

_INTERPRET = False


def _router_body(x_ref, gw_ref, logits_ref, ids_ref, wts_ref):
    x = x_ref[...]
    logits = lax.dot_general(x, gw_ref[...], (((1,), (1,)), ((), ())),
                             preferred_element_type=jnp.float32)
    logits_ref[...] = logits
    m = jnp.max(logits, axis=-1, keepdims=True)
    p = jnp.exp(logits - m)
    p = p / jnp.sum(p, axis=-1, keepdims=True)
    iot = lax.broadcasted_iota(jnp.int32, p.shape, 1)
    m0 = jnp.max(p, axis=-1, keepdims=True)
    i0 = jnp.min(jnp.where(p == m0, iot, E), axis=-1, keepdims=True)
    p2 = jnp.where(iot == i0, -jnp.inf, p)
    m1 = jnp.max(p2, axis=-1, keepdims=True)
    i1 = jnp.min(jnp.where(p2 == m1, iot, E), axis=-1, keepdims=True)
    s = m0 + m1
    ids_ref[...] = jnp.concatenate([i0, i1], axis=-1)
    wts_ref[...] = jnp.concatenate([m0 / s, m1 / s], axis=-1)


def _router(hidden_states, gate_w):
    return pl.pallas_call(
        _router_body,
        grid=(T // BT,),
        in_specs=[
            pl.BlockSpec((BT, HIDDEN), lambda i: (i, 0)),
            pl.BlockSpec((E, HIDDEN), lambda i: (0, 0)),
        ],
        out_specs=[
            pl.BlockSpec((BT, E), lambda i: (i, 0)),
            pl.BlockSpec((BT, TOPK), lambda i: (i, 0)),
            pl.BlockSpec((BT, TOPK), lambda i: (i, 0)),
        ],
        out_shape=[
            jax.ShapeDtypeStruct((T, E), jnp.float32),
            jax.ShapeDtypeStruct((T, TOPK), jnp.int32),
            jax.ShapeDtypeStruct((T, TOPK), jnp.float32),
        ],
        interpret=_INTERPRET,
    )(hidden_states, gate_w)


def _dispatch_indices(ids, wts):
    i32 = jnp.int32
    e_flat = ids.reshape(-1).astype(i32)
    w_flat = wts.reshape(-1)
    onehot = (e_flat[:, None] == jnp.arange(E, dtype=i32)[None, :]).astype(i32)
    cum = jnp.cumsum(onehot, axis=0)
    rank = jnp.sum(onehot * cum, axis=1) - 1
    counts = cum[-1]
    padded = ((counts + BS - 1) // BS) * BS
    pend = jnp.cumsum(padded)
    poff = pend - padded
    nr = (pend[-1] // BS).astype(i32)

    bstarts = jnp.arange(NB, dtype=i32) * BS
    be_raw = jnp.minimum(jnp.sum(pend[None, :] <= bstarts[:, None], axis=1),
                         E - 1).astype(i32)
    be = jnp.where(jnp.arange(NB, dtype=i32) < nr, be_raw,
                   be_raw[jnp.maximum(nr - 1, 0)])

    dest = jnp.sum(onehot * poff[None, :], axis=1) + rank
    tok = jnp.arange(S, dtype=i32) // TOPK
    tok_p = jnp.zeros((S_MAX,), i32).at[dest].set(tok)
    wt_p = jnp.zeros((S_MAX,), jnp.float32).at[dest].set(w_flat)
    pos = dest.reshape(T, TOPK)
    return tok_p, wt_p, be, nr, pos[:, 0], pos[:, 1]


def _ffn_body(be_ref, nr_ref, xs_ref, w1_ref, w2_ref, wc_ref, ys_ref):
    i = pl.program_id(0)

    @pl.when(i < nr_ref[0])
    def _():
        x = xs_ref[...].astype(jnp.bfloat16)
        gu = lax.dot_general(x, w1_ref[0].astype(jnp.bfloat16),
                             (((1,), (1,)), ((), ())),
                             preferred_element_type=jnp.float32)
        g = gu[:, :FFN]
        u = gu[:, FFN:]
        h = g * jax.nn.sigmoid(g) * u
        hw = (h * wc_ref[:, 0:1]).astype(jnp.bfloat16)
        ys_ref[...] = lax.dot_general(hw, w2_ref[0].astype(jnp.bfloat16),
                                      (((1,), (1,)), ((), ())),
                                      preferred_element_type=jnp.float32)


def _ffn(xs, w1, w2, wcol, be, nr):
    grid_spec = pltpu.PrefetchScalarGridSpec(
        num_scalar_prefetch=2,
        grid=(NB,),
        in_specs=[
            pl.BlockSpec((BS, HIDDEN), lambda i, be, nr: (i, 0)),
            pl.BlockSpec((1, 2 * FFN, HIDDEN), lambda i, be, nr: (be[i], 0, 0)),
            pl.BlockSpec((1, HIDDEN, FFN), lambda i, be, nr: (be[i], 0, 0)),
            pl.BlockSpec((BS, 128), lambda i, be, nr: (i, 0)),
        ],
        out_specs=pl.BlockSpec((BS, HIDDEN), lambda i, be, nr: (i, 0)),
    )
    return pl.pallas_call(
        _ffn_body,
        grid_spec=grid_spec,
        out_shape=jax.ShapeDtypeStruct((S_MAX, HIDDEN), jnp.float32),
        interpret=_INTERPRET,
    )(be, nr, xs, w1, w2, wcol)


def kernel(hidden_states, gate_w, w1, w2):
    router_logits, ids, wts = _router(hidden_states, gate_w)
    tok_p, wt_p, be, nr, pos0, pos1 = _dispatch_indices(ids, wts)

    xs = hidden_states[tok_p]
    wcol = jnp.broadcast_to(wt_p[:, None], (S_MAX, 128))

    ysw = _ffn(xs, w1, w2, wcol, be, nr[None])

    final = ysw[pos0] + ysw[pos1]
    return final, router_logits
